# baseline (speedup 1.0000x reference)
import jax
import jax.numpy as jnp
from jax.experimental import pallas as pl

B, C, H, W = 2, 96, 224, 224
HW = H * W
LIMIT = 20
EPS = 1e-5
TH = 32
HT = H // TH


def _shift(a, axis, d):
    if d == 0:
        return a
    n = a.shape[axis]
    zshape = list(a.shape)
    zshape[axis] = abs(d)
    zeros = jnp.zeros(zshape, a.dtype)
    if d > 0:
        sl = jax.lax.slice_in_dim(a, 0, n - d, axis=axis)
        return jnp.concatenate([zeros, sl], axis=axis)
    sl = jax.lax.slice_in_dim(a, -d, n, axis=axis)
    return jnp.concatenate([sl, zeros], axis=axis)


def _conv_kernel(xa_ref, xh_ref, w_ref, b_ref, attn_ref, mask_ref):
    acc = jnp.zeros((3, TH, W), jnp.float32)
    xx = jnp.concatenate([xa_ref[0], xh_ref[0]], axis=1)
    for dy in range(3):
        for kx in range(3):
            tap = xx[:, dy:dy + TH, kx:kx + W]
            wk = w_ref[:, :, dy, kx]
            acc = acc + jax.lax.dot_general(
                wk, tap, (((1,), (0,)), ((), ())),
                preferred_element_type=jnp.float32)
    acc = acc + b_ref[:][:, :, None]
    mx = jnp.max(acc, axis=0, keepdims=True)
    e = jnp.exp(acc - mx)
    attn_ref[0] = e / jnp.sum(e, axis=0, keepdims=True)
    o0, o1, o2 = acc[0], acc[1], acc[2]
    am1 = (o1 > o0) & (o1 >= o2)
    am2 = (o2 > o0) & (o2 > o1)
    mask_ref[0, 0] = am1.astype(jnp.int32)
    mask_ref[0, 1] = am2.astype(jnp.int32)


def _cc_kernel(mask_ref, inv_ref, meta_ref):
    mi = mask_ref[:]
    m = mi != 0
    ih = jax.lax.broadcasted_iota(jnp.int32, (4, H, W), 1)
    iw = jax.lax.broadcasted_iota(jnp.int32, (4, H, W), 2)
    iota = ih * W + iw + 1
    lab0 = jnp.where(m, iota, 0)

    def sweep(lab):
        okf = mi * _shift(mi, 2, 1)
        okb = mi * _shift(mi, 2, -1)
        d = 1
        while d < W:
            lab = jnp.maximum(lab, jnp.where(okf != 0, _shift(lab, 2, d), 0))
            lab = jnp.maximum(lab, jnp.where(okb != 0, _shift(lab, 2, -d), 0))
            okf = okf * _shift(okf, 2, d)
            okb = okb * _shift(okb, 2, -d)
            d *= 2
        okf = mi * _shift(mi, 1, 1)
        okb = mi * _shift(mi, 1, -1)
        d = 1
        while d < H:
            lab = jnp.maximum(lab, jnp.where(okf != 0, _shift(lab, 1, d), 0))
            lab = jnp.maximum(lab, jnp.where(okb != 0, _shift(lab, 1, -d), 0))
            okf = okf * _shift(okf, 1, d)
            okb = okb * _shift(okb, 1, -d)
            d *= 2
        return lab

    def fixpoint(lab):
        state = (lab, sweep(lab))
        state = jax.lax.while_loop(
            lambda s: jnp.any(s[0] != s[1]),
            lambda s: (s[1], sweep(s[1])),
            state)
        return state[1]

    lab = fixpoint(lab0)

    is_head = m & (lab == iota)
    heads = is_head.astype(jnp.int32)
    cs = heads
    d = 1
    while d < W:
        cs = cs + _shift(cs, 2, d)
        d *= 2
    row_tot = cs[:, :, W - 1:W]
    rc = row_tot
    d = 1
    while d < H:
        rc = rc + _shift(rc, 1, d)
        d *= 2
    excl_rows = rc - row_tot
    incl = cs + excl_rows
    excl = incl - heads
    has_bg = jnp.any(jnp.any(~m, axis=2), axis=1)
    bg_i = has_bg.astype(jnp.int32)
    rank_at_head = excl + bg_i[:, None, None]
    seed = jnp.where(is_head, rank_at_head, 0)
    inv_ref[:] = fixpoint(seed)

    n_vec = incl[:, H - 1, W - 1] + bg_i
    col = jnp.concatenate([n_vec, bg_i], axis=0).reshape(8, 1)
    meta_ref[:] = jnp.concatenate(
        [col, jnp.zeros((8, 127), jnp.int32)], axis=1)


def _segsum_kernel(x_ref, inv_ref, sums_ref):
    @pl.when(pl.program_id(1) == 0)
    def _():
        sums_ref[...] = jnp.zeros_like(sums_ref)

    x = x_ref[0].reshape(C, TH * W)
    rseg = jax.lax.broadcasted_iota(jnp.int32, (LIMIT, TH * W), 0)
    for s in range(2):
        inv = inv_ref[0, s].reshape(1, TH * W)
        oh = (inv == rseg).astype(jnp.float32)
        part = jax.lax.dot_general(oh, x, (((1,), (1,)), ((), ())),
                                   preferred_element_type=jnp.float32)
        cnt = jnp.sum(oh, axis=1, keepdims=True)
        sums_ref[0, s] += jnp.concatenate([part, cnt], axis=1)


def _factors_kernel(sums_ref, n_ref, bg_ref, mv_ref):
    s1 = sums_ref[0, 0]
    s2 = sums_ref[0, 1]
    means1 = s1[:, :C] / jnp.maximum(s1[:, C:C + 1], 1.0)
    means2 = s2[:, :C] / jnp.maximum(s2[:, C:C + 1], 1.0)
    mm = jax.lax.dot_general(means1[1:], means2[1:], (((1,), (1,)), ((), ())),
                             preferred_element_type=jnp.float32)
    n1 = n_ref[0, 0, 0]
    n2 = n_ref[0, 0, 1]
    k1 = jnp.minimum(LIMIT, n1)
    k2 = jnp.minimum(LIMIT, n2)
    ri = jax.lax.broadcasted_iota(jnp.int32, (LIMIT - 1, LIMIT - 1), 0)
    rj = jax.lax.broadcasted_iota(jnp.int32, (LIMIT - 1, LIMIT - 1), 1)
    valid = (ri < k1 - 1) & (rj < k2 - 1)
    sub = 1.0 + jnp.where(valid, mm, 0.0)
    m1 = jnp.ones((LIMIT - 1, 1), jnp.float32)
    m2 = jnp.ones((1, LIMIT - 1), jnp.float32)
    for j in range(LIMIT - 1):
        m1 = m1 * sub[:, j:j + 1]
        m2 = m2 * sub[j:j + 1, :]
    rr = jax.lax.broadcasted_iota(jnp.int32, (1, LIMIT), 1)
    one = jnp.ones((1, 1), jnp.float32)
    m1p = jnp.concatenate([one, m1.reshape(1, LIMIT - 1)], axis=1)
    m2p = jnp.concatenate([one, m2], axis=1)
    apply = (n1 > 1) & (n2 > 1) & (bg_ref[0, 0, 0] > 0) & (bg_ref[0, 0, 1] > 0)
    mvec1 = jnp.where(apply & (rr >= 1) & (rr < k1), m1p, 1.0)
    mvec2 = jnp.where(apply & (rr >= 1) & (rr < k2), m2p, 1.0)
    mv_ref[0, 0] = mvec1[0]
    mv_ref[0, 1] = mvec2[0]


def _mult_kernel(x_ref, inv_ref, mv_ref, mult_ref, stats_ref):
    @pl.when(pl.program_id(1) == 0)
    def _():
        stats_ref[...] = jnp.zeros_like(stats_ref)

    x = x_ref[0].reshape(C, TH * W)
    rseg = jax.lax.broadcasted_iota(jnp.int32, (LIMIT, TH * W), 0)
    mult = None
    for s in range(2):
        inv = inv_ref[0, s].reshape(1, TH * W)
        oh = (inv == rseg).astype(jnp.float32)
        g = jax.lax.dot_general(mv_ref[0, s].reshape(1, LIMIT), oh,
                                (((1,), (0,)), ((), ())),
                                preferred_element_type=jnp.float32)
        ms = jnp.where(inv < LIMIT, g, 1.0)
        mult = ms if mult is None else mult * ms
    mult_ref[0] = mult.reshape(TH, W)
    xu = x * mult
    stats_ref[0, 0] += jnp.sum(xu, axis=1)
    stats_ref[0, 1] += jnp.sum(xu * xu, axis=1)


def _norm_kernel(x_ref, mult_ref, scale_ref, shift_ref, out_ref):
    xm = x_ref[0] * mult_ref[0][None, :, :]
    out_ref[0] = xm * scale_ref[:][:, :, None] + shift_ref[:][:, :, None]


def kernel(x, conv_w, conv_b, gamma, beta):
    xp = jnp.pad(x, ((0, 0), (0, 0), (1, 31), (1, 1)))
    xhalo = xp[:, :, 32:256].reshape(B, C, HT, 32, W + 2)[:, :, :, :2]
    xhalo = xhalo.transpose(0, 2, 1, 3, 4).reshape(B * HT, C, 2, W + 2)

    attn, masks = pl.pallas_call(
        _conv_kernel,
        grid=(B, HT),
        in_specs=[
            pl.BlockSpec((1, C, TH, W + 2), lambda b, h: (b, 0, h, 0)),
            pl.BlockSpec((1, C, 2, W + 2), lambda b, h: (b * HT + h, 0, 0, 0)),
            pl.BlockSpec((3, C, 3, 3), lambda b, h: (0, 0, 0, 0)),
            pl.BlockSpec((3, 1), lambda b, h: (0, 0)),
        ],
        out_specs=[
            pl.BlockSpec((1, 3, TH, W), lambda b, h: (b, 0, h, 0)),
            pl.BlockSpec((1, 2, TH, W), lambda b, h: (b, 0, h, 0)),
        ],
        out_shape=[
            jax.ShapeDtypeStruct((B, 3, H, W), jnp.float32),
            jax.ShapeDtypeStruct((B, 2, H, W), jnp.int32),
        ],
    )(xp, xhalo, conv_w, conv_b.reshape(3, 1))

    inv, meta = pl.pallas_call(
        _cc_kernel,
        out_shape=[
            jax.ShapeDtypeStruct((2 * B, H, W), jnp.int32),
            jax.ShapeDtypeStruct((8, 128), jnp.int32),
        ],
    )(masks.reshape(2 * B, H, W))
    inv = inv.reshape(B, 2, H, W)
    n_arr = meta[0:4, 0].reshape(B, 1, 2)
    bg_arr = meta[4:8, 0].reshape(B, 1, 2)

    sums = pl.pallas_call(
        _segsum_kernel,
        grid=(B, HT),
        in_specs=[
            pl.BlockSpec((1, C, TH, W), lambda b, h: (b, 0, h, 0)),
            pl.BlockSpec((1, 2, TH, W), lambda b, h: (b, 0, h, 0)),
        ],
        out_specs=pl.BlockSpec((1, 2, LIMIT, C + 1), lambda b, h: (b, 0, 0, 0)),
        out_shape=jax.ShapeDtypeStruct((B, 2, LIMIT, C + 1), jnp.float32),
    )(x, inv)

    mv = pl.pallas_call(
        _factors_kernel,
        grid=(B,),
        in_specs=[
            pl.BlockSpec((1, 2, LIMIT, C + 1), lambda b: (b, 0, 0, 0)),
            pl.BlockSpec((1, 1, 2), lambda b: (b, 0, 0)),
            pl.BlockSpec((1, 1, 2), lambda b: (b, 0, 0)),
        ],
        out_specs=pl.BlockSpec((1, 2, LIMIT), lambda b: (b, 0, 0)),
        out_shape=jax.ShapeDtypeStruct((B, 2, LIMIT), jnp.float32),
    )(sums, n_arr, bg_arr)

    mult, stats = pl.pallas_call(
        _mult_kernel,
        grid=(B, HT),
        in_specs=[
            pl.BlockSpec((1, C, TH, W), lambda b, h: (b, 0, h, 0)),
            pl.BlockSpec((1, 2, TH, W), lambda b, h: (b, 0, h, 0)),
            pl.BlockSpec((1, 2, LIMIT), lambda b, h: (b, 0, 0)),
        ],
        out_specs=[
            pl.BlockSpec((1, TH, W), lambda b, h: (b, h, 0)),
            pl.BlockSpec((1, 2, C), lambda b, h: (b, 0, 0)),
        ],
        out_shape=[
            jax.ShapeDtypeStruct((B, H, W), jnp.float32),
            jax.ShapeDtypeStruct((B, 2, C), jnp.float32),
        ],
    )(x, inv, mv)

    total = jnp.sum(stats, axis=0)
    n_el = jnp.float32(B * HW)
    mean = total[0] / n_el
    var = total[1] / n_el - mean * mean
    scale = gamma / jnp.sqrt(var + EPS)
    shift = beta - mean * scale

    xn = pl.pallas_call(
        _norm_kernel,
        grid=(B, HT),
        in_specs=[
            pl.BlockSpec((1, C, TH, W), lambda b, h: (b, 0, h, 0)),
            pl.BlockSpec((1, TH, W), lambda b, h: (b, h, 0)),
            pl.BlockSpec((C, 1), lambda b, h: (0, 0)),
            pl.BlockSpec((C, 1), lambda b, h: (0, 0)),
        ],
        out_specs=pl.BlockSpec((1, C, TH, W), lambda b, h: (b, 0, h, 0)),
        out_shape=jax.ShapeDtypeStruct((B, C, H, W), jnp.float32),
    )(x, mult, scale.reshape(C, 1), shift.reshape(C, 1))

    return (xn, attn)


# speedup vs baseline: 1.2205x; 1.2205x over previous
import jax
import jax.numpy as jnp
from jax.experimental import pallas as pl

B, C, H, W = 2, 96, 224, 224
HW = H * W
LIMIT = 20
EPS = 1e-5
TH = 32
HT = H // TH


def _shift(a, axis, d):
    if d == 0:
        return a
    n = a.shape[axis]
    zshape = list(a.shape)
    zshape[axis] = abs(d)
    zeros = jnp.zeros(zshape, a.dtype)
    if d > 0:
        sl = jax.lax.slice_in_dim(a, 0, n - d, axis=axis)
        return jnp.concatenate([zeros, sl], axis=axis)
    sl = jax.lax.slice_in_dim(a, -d, n, axis=axis)
    return jnp.concatenate([sl, zeros], axis=axis)


def _conv_kernel(xa_ref, xb_ref, w_ref, b_ref, attn_ref, mask_ref):
    acc = jnp.zeros((3, TH, W), jnp.float32)
    xx = jnp.concatenate([xa_ref[0], xb_ref[0][:, 0:2]], axis=1)
    for dy in range(3):
        for kx in range(3):
            tap = xx[:, dy:dy + TH, kx:kx + W]
            wk = w_ref[:, :, dy, kx]
            acc = acc + jax.lax.dot_general(
                wk, tap, (((1,), (0,)), ((), ())),
                preferred_element_type=jnp.float32)
    acc = acc + b_ref[:][:, :, None]
    mx = jnp.max(acc, axis=0, keepdims=True)
    e = jnp.exp(acc - mx)
    attn_ref[0] = e / jnp.sum(e, axis=0, keepdims=True)
    o0, o1, o2 = acc[0], acc[1], acc[2]
    am1 = (o1 > o0) & (o1 >= o2)
    am2 = (o2 > o0) & (o2 > o1)
    mask_ref[0, 0] = am1.astype(jnp.int32)
    mask_ref[0, 1] = am2.astype(jnp.int32)


def _cc_kernel(mask_ref, inv_ref, meta_ref):
    mi = mask_ref[:]
    m = mi != 0
    ih = jax.lax.broadcasted_iota(jnp.int32, (4, H, W), 1)
    iw = jax.lax.broadcasted_iota(jnp.int32, (4, H, W), 2)
    iota = ih * W + iw + 1
    lab0 = jnp.where(m, iota, 0)

    def sweep(lab):
        okf = mi * _shift(mi, 2, 1)
        okb = mi * _shift(mi, 2, -1)
        d = 1
        while d < W:
            lab = jnp.maximum(lab, jnp.where(okf != 0, _shift(lab, 2, d), 0))
            lab = jnp.maximum(lab, jnp.where(okb != 0, _shift(lab, 2, -d), 0))
            okf = okf * _shift(okf, 2, d)
            okb = okb * _shift(okb, 2, -d)
            d *= 2
        okf = mi * _shift(mi, 1, 1)
        okb = mi * _shift(mi, 1, -1)
        d = 1
        while d < H:
            lab = jnp.maximum(lab, jnp.where(okf != 0, _shift(lab, 1, d), 0))
            lab = jnp.maximum(lab, jnp.where(okb != 0, _shift(lab, 1, -d), 0))
            okf = okf * _shift(okf, 1, d)
            okb = okb * _shift(okb, 1, -d)
            d *= 2
        return lab

    def fixpoint(lab):
        state = (lab, sweep(lab))
        state = jax.lax.while_loop(
            lambda s: jnp.any(s[0] != s[1]),
            lambda s: (s[1], sweep(s[1])),
            state)
        return state[1]

    lab = fixpoint(lab0)

    is_head = m & (lab == iota)
    heads = is_head.astype(jnp.int32)
    cs = heads
    d = 1
    while d < W:
        cs = cs + _shift(cs, 2, d)
        d *= 2
    row_tot = cs[:, :, W - 1:W]
    rc = row_tot
    d = 1
    while d < H:
        rc = rc + _shift(rc, 1, d)
        d *= 2
    excl_rows = rc - row_tot
    incl = cs + excl_rows
    excl = incl - heads
    has_bg = jnp.any(jnp.any(~m, axis=2), axis=1)
    bg_i = has_bg.astype(jnp.int32)

    n_heads = incl[:, H - 1, W - 1]
    pos = iota - 1
    inv_cnt = jnp.zeros((4, H, W), jnp.int32)
    for r in range(LIMIT + 1):
        sel = is_head & (excl == r)
        t_sum = jnp.sum(jnp.sum(jnp.where(sel, pos, 0), axis=2), axis=1)
        t_r = jnp.where(r < n_heads, t_sum, HW + 5)
        inv_cnt = inv_cnt + (t_r[:, None, None] < lab - 1).astype(jnp.int32)
    inv_ref[:] = jnp.where(m, bg_i[:, None, None] + inv_cnt, 0)

    n_vec = n_heads + bg_i
    col = jnp.concatenate([n_vec, bg_i], axis=0).reshape(8, 1)
    meta_ref[:] = jnp.concatenate(
        [col, jnp.zeros((8, 127), jnp.int32)], axis=1)


def _segsum_kernel(x_ref, inv_ref, sums_ref):
    @pl.when(pl.program_id(1) == 0)
    def _():
        sums_ref[...] = jnp.zeros_like(sums_ref)

    x = x_ref[0].reshape(C, TH * W)
    rseg = jax.lax.broadcasted_iota(jnp.int32, (LIMIT, TH * W), 0)
    for s in range(2):
        inv = inv_ref[0, s].reshape(1, TH * W)
        oh = (inv == rseg).astype(jnp.float32)
        part = jax.lax.dot_general(oh, x, (((1,), (1,)), ((), ())),
                                   preferred_element_type=jnp.float32)
        cnt = jnp.sum(oh, axis=1, keepdims=True)
        sums_ref[0, s] += jnp.concatenate([part, cnt], axis=1)


def _factors_kernel(sums_ref, n_ref, bg_ref, mv_ref):
    s1 = sums_ref[0, 0]
    s2 = sums_ref[0, 1]
    means1 = s1[:, :C] / jnp.maximum(s1[:, C:C + 1], 1.0)
    means2 = s2[:, :C] / jnp.maximum(s2[:, C:C + 1], 1.0)
    mm = jax.lax.dot_general(means1[1:], means2[1:], (((1,), (1,)), ((), ())),
                             preferred_element_type=jnp.float32)
    n1 = n_ref[0, 0, 0]
    n2 = n_ref[0, 0, 1]
    k1 = jnp.minimum(LIMIT, n1)
    k2 = jnp.minimum(LIMIT, n2)
    ri = jax.lax.broadcasted_iota(jnp.int32, (LIMIT - 1, LIMIT - 1), 0)
    rj = jax.lax.broadcasted_iota(jnp.int32, (LIMIT - 1, LIMIT - 1), 1)
    valid = (ri < k1 - 1) & (rj < k2 - 1)
    sub = 1.0 + jnp.where(valid, mm, 0.0)
    m1 = jnp.ones((LIMIT - 1, 1), jnp.float32)
    m2 = jnp.ones((1, LIMIT - 1), jnp.float32)
    for j in range(LIMIT - 1):
        m1 = m1 * sub[:, j:j + 1]
        m2 = m2 * sub[j:j + 1, :]
    rr = jax.lax.broadcasted_iota(jnp.int32, (1, LIMIT), 1)
    one = jnp.ones((1, 1), jnp.float32)
    m1p = jnp.concatenate([one, m1.reshape(1, LIMIT - 1)], axis=1)
    m2p = jnp.concatenate([one, m2], axis=1)
    apply = (n1 > 1) & (n2 > 1) & (bg_ref[0, 0, 0] > 0) & (bg_ref[0, 0, 1] > 0)
    mvec1 = jnp.where(apply & (rr >= 1) & (rr < k1), m1p, 1.0)
    mvec2 = jnp.where(apply & (rr >= 1) & (rr < k2), m2p, 1.0)
    mv_ref[0, 0] = mvec1[0]
    mv_ref[0, 1] = mvec2[0]


def _mult_kernel(x_ref, inv_ref, mv_ref, mult_ref, stats_ref):
    @pl.when(pl.program_id(1) == 0)
    def _():
        stats_ref[...] = jnp.zeros_like(stats_ref)

    x = x_ref[0].reshape(C, TH * W)
    rseg = jax.lax.broadcasted_iota(jnp.int32, (LIMIT, TH * W), 0)
    mult = None
    for s in range(2):
        inv = inv_ref[0, s].reshape(1, TH * W)
        oh = (inv == rseg).astype(jnp.float32)
        g = jax.lax.dot_general(mv_ref[0, s].reshape(1, LIMIT), oh,
                                (((1,), (0,)), ((), ())),
                                preferred_element_type=jnp.float32)
        ms = jnp.where(inv < LIMIT, g, 1.0)
        mult = ms if mult is None else mult * ms
    mult_ref[0] = mult.reshape(TH, W)
    xu = x * mult
    stats_ref[0, 0] += jnp.sum(xu, axis=1)
    stats_ref[0, 1] += jnp.sum(xu * xu, axis=1)


def _norm_kernel(x_ref, mult_ref, scale_ref, shift_ref, out_ref):
    xm = x_ref[0] * mult_ref[0][None, :, :]
    out_ref[0] = xm * scale_ref[:][:, :, None] + shift_ref[:][:, :, None]


def kernel(x, conv_w, conv_b, gamma, beta):
    xp = jnp.pad(x, ((0, 0), (0, 0), (1, 31), (1, 1)))

    attn, masks = pl.pallas_call(
        _conv_kernel,
        grid=(B, HT),
        in_specs=[
            pl.BlockSpec((1, C, TH, W + 2), lambda b, h: (b, 0, h, 0)),
            pl.BlockSpec((1, C, TH, W + 2), lambda b, h: (b, 0, h + 1, 0)),
            pl.BlockSpec((3, C, 3, 3), lambda b, h: (0, 0, 0, 0)),
            pl.BlockSpec((3, 1), lambda b, h: (0, 0)),
        ],
        out_specs=[
            pl.BlockSpec((1, 3, TH, W), lambda b, h: (b, 0, h, 0)),
            pl.BlockSpec((1, 2, TH, W), lambda b, h: (b, 0, h, 0)),
        ],
        out_shape=[
            jax.ShapeDtypeStruct((B, 3, H, W), jnp.float32),
            jax.ShapeDtypeStruct((B, 2, H, W), jnp.int32),
        ],
    )(xp, xp, conv_w, conv_b.reshape(3, 1))

    inv, meta = pl.pallas_call(
        _cc_kernel,
        out_shape=[
            jax.ShapeDtypeStruct((2 * B, H, W), jnp.int32),
            jax.ShapeDtypeStruct((8, 128), jnp.int32),
        ],
    )(masks.reshape(2 * B, H, W))
    inv = inv.reshape(B, 2, H, W)
    n_arr = meta[0:4, 0].reshape(B, 1, 2)
    bg_arr = meta[4:8, 0].reshape(B, 1, 2)

    sums = pl.pallas_call(
        _segsum_kernel,
        grid=(B, HT),
        in_specs=[
            pl.BlockSpec((1, C, TH, W), lambda b, h: (b, 0, h, 0)),
            pl.BlockSpec((1, 2, TH, W), lambda b, h: (b, 0, h, 0)),
        ],
        out_specs=pl.BlockSpec((1, 2, LIMIT, C + 1), lambda b, h: (b, 0, 0, 0)),
        out_shape=jax.ShapeDtypeStruct((B, 2, LIMIT, C + 1), jnp.float32),
    )(x, inv)

    mv = pl.pallas_call(
        _factors_kernel,
        grid=(B,),
        in_specs=[
            pl.BlockSpec((1, 2, LIMIT, C + 1), lambda b: (b, 0, 0, 0)),
            pl.BlockSpec((1, 1, 2), lambda b: (b, 0, 0)),
            pl.BlockSpec((1, 1, 2), lambda b: (b, 0, 0)),
        ],
        out_specs=pl.BlockSpec((1, 2, LIMIT), lambda b: (b, 0, 0)),
        out_shape=jax.ShapeDtypeStruct((B, 2, LIMIT), jnp.float32),
    )(sums, n_arr, bg_arr)

    mult, stats = pl.pallas_call(
        _mult_kernel,
        grid=(B, HT),
        in_specs=[
            pl.BlockSpec((1, C, TH, W), lambda b, h: (b, 0, h, 0)),
            pl.BlockSpec((1, 2, TH, W), lambda b, h: (b, 0, h, 0)),
            pl.BlockSpec((1, 2, LIMIT), lambda b, h: (b, 0, 0)),
        ],
        out_specs=[
            pl.BlockSpec((1, TH, W), lambda b, h: (b, h, 0)),
            pl.BlockSpec((1, 2, C), lambda b, h: (b, 0, 0)),
        ],
        out_shape=[
            jax.ShapeDtypeStruct((B, H, W), jnp.float32),
            jax.ShapeDtypeStruct((B, 2, C), jnp.float32),
        ],
    )(x, inv, mv)

    total = jnp.sum(stats, axis=0)
    n_el = jnp.float32(B * HW)
    mean = total[0] / n_el
    var = total[1] / n_el - mean * mean
    scale = gamma / jnp.sqrt(var + EPS)
    shift = beta - mean * scale

    xn = pl.pallas_call(
        _norm_kernel,
        grid=(B, HT),
        in_specs=[
            pl.BlockSpec((1, C, TH, W), lambda b, h: (b, 0, h, 0)),
            pl.BlockSpec((1, TH, W), lambda b, h: (b, h, 0)),
            pl.BlockSpec((C, 1), lambda b, h: (0, 0)),
            pl.BlockSpec((C, 1), lambda b, h: (0, 0)),
        ],
        out_specs=pl.BlockSpec((1, C, TH, W), lambda b, h: (b, 0, h, 0)),
        out_shape=jax.ShapeDtypeStruct((B, C, H, W), jnp.float32),
    )(x, mult, scale.reshape(C, 1), shift.reshape(C, 1))

    return (xn, attn)


# speedup vs baseline: 1.2243x; 1.0031x over previous
import jax
import jax.numpy as jnp
from jax.experimental import pallas as pl

B, C, H, W = 2, 96, 224, 224
HW = H * W
LIMIT = 20
EPS = 1e-5
TH = 32
HT = H // TH


def _shift(a, axis, d):
    if d == 0:
        return a
    n = a.shape[axis]
    zshape = list(a.shape)
    zshape[axis] = abs(d)
    zeros = jnp.zeros(zshape, a.dtype)
    if d > 0:
        sl = jax.lax.slice_in_dim(a, 0, n - d, axis=axis)
        return jnp.concatenate([zeros, sl], axis=axis)
    sl = jax.lax.slice_in_dim(a, -d, n, axis=axis)
    return jnp.concatenate([sl, zeros], axis=axis)


def _conv_kernel(xa_ref, xb_ref, w_ref, b_ref, attn_ref, mask_ref):
    acc = jnp.zeros((3, TH, W), jnp.float32)
    xx = jnp.concatenate([xa_ref[0], xb_ref[0][:, 0:2]], axis=1)
    for dy in range(3):
        for kx in range(3):
            tap = xx[:, dy:dy + TH, kx:kx + W]
            wk = w_ref[:, :, dy, kx]
            acc = acc + jax.lax.dot_general(
                wk, tap, (((1,), (0,)), ((), ())),
                preferred_element_type=jnp.float32)
    acc = acc + b_ref[:][:, :, None]
    mx = jnp.max(acc, axis=0, keepdims=True)
    e = jnp.exp(acc - mx)
    attn_ref[0] = e / jnp.sum(e, axis=0, keepdims=True)
    o0, o1, o2 = acc[0], acc[1], acc[2]
    am1 = (o1 > o0) & (o1 >= o2)
    am2 = (o2 > o0) & (o2 > o1)
    mask_ref[0, 0] = am1.astype(jnp.int32)
    mask_ref[0, 1] = am2.astype(jnp.int32)


def _cc_kernel(mask_ref, inv_ref, meta_ref):
    mi = mask_ref[:]
    m = mi != 0
    ih = jax.lax.broadcasted_iota(jnp.int32, (4, H, W), 1)
    iw = jax.lax.broadcasted_iota(jnp.int32, (4, H, W), 2)
    iota = ih * W + iw + 1
    lab0 = jnp.where(m, iota, 0)

    def sweep(lab):
        okf = mi * _shift(mi, 2, 1)
        okb = mi * _shift(mi, 2, -1)
        d = 1
        while d < W:
            lab = jnp.maximum(lab, jnp.where(okf != 0, _shift(lab, 2, d), 0))
            lab = jnp.maximum(lab, jnp.where(okb != 0, _shift(lab, 2, -d), 0))
            okf = okf * _shift(okf, 2, d)
            okb = okb * _shift(okb, 2, -d)
            d *= 2
        okf = mi * _shift(mi, 1, 1)
        okb = mi * _shift(mi, 1, -1)
        d = 1
        while d < H:
            lab = jnp.maximum(lab, jnp.where(okf != 0, _shift(lab, 1, d), 0))
            lab = jnp.maximum(lab, jnp.where(okb != 0, _shift(lab, 1, -d), 0))
            okf = okf * _shift(okf, 1, d)
            okb = okb * _shift(okb, 1, -d)
            d *= 2
        return lab

    def fixpoint(lab):
        state = (lab, sweep(lab))
        state = jax.lax.while_loop(
            lambda s: jnp.any(s[0] != s[1]),
            lambda s: (s[1], sweep(s[1])),
            state)
        return state[1]

    lab = fixpoint(lab0)

    is_head = m & (lab == iota)
    heads = is_head.astype(jnp.int32)
    cs = heads
    d = 1
    while d < W:
        cs = cs + _shift(cs, 2, d)
        d *= 2
    row_tot = cs[:, :, W - 1:W]
    rc = row_tot
    d = 1
    while d < H:
        rc = rc + _shift(rc, 1, d)
        d *= 2
    excl_rows = rc - row_tot
    incl = cs + excl_rows
    excl = incl - heads
    has_bg = jnp.any(jnp.any(~m, axis=2), axis=1)
    bg_i = has_bg.astype(jnp.int32)

    n_heads = incl[:, H - 1, W - 1]
    pos = iota - 1
    inv_cnt = jnp.zeros((4, H, W), jnp.int32)
    for r in range(LIMIT + 1):
        sel = is_head & (excl == r)
        t_sum = jnp.sum(jnp.sum(jnp.where(sel, pos, 0), axis=2), axis=1)
        t_r = jnp.where(r < n_heads, t_sum, HW + 5)
        inv_cnt = inv_cnt + (t_r[:, None, None] < lab - 1).astype(jnp.int32)
    inv_ref[:] = jnp.where(m, bg_i[:, None, None] + inv_cnt, 0)

    n_vec = n_heads + bg_i
    col = jnp.concatenate([n_vec, bg_i], axis=0).reshape(8, 1)
    meta_ref[:] = jnp.concatenate(
        [col, jnp.zeros((8, 127), jnp.int32)], axis=1)


def _segsum_kernel(x_ref, inv_ref, n_ref, bg_ref, sums_ref, mv_ref):
    @pl.when(pl.program_id(1) == 0)
    def _():
        sums_ref[...] = jnp.zeros_like(sums_ref)

    x = x_ref[0].reshape(C, TH * W)
    rseg = jax.lax.broadcasted_iota(jnp.int32, (LIMIT, TH * W), 0)
    for s in range(2):
        inv = inv_ref[0, s].reshape(1, TH * W)
        oh = (inv == rseg).astype(jnp.float32)
        part = jax.lax.dot_general(oh, x, (((1,), (1,)), ((), ())),
                                   preferred_element_type=jnp.float32)
        cnt = jnp.sum(oh, axis=1, keepdims=True)
        sums_ref[0, s] += jnp.concatenate([part, cnt], axis=1)

    @pl.when(pl.program_id(1) == HT - 1)
    def _():
        _factors_body(sums_ref, n_ref, bg_ref, mv_ref)


def _factors_body(sums_ref, n_ref, bg_ref, mv_ref):
    s1 = sums_ref[0, 0]
    s2 = sums_ref[0, 1]
    means1 = s1[:, :C] / jnp.maximum(s1[:, C:C + 1], 1.0)
    means2 = s2[:, :C] / jnp.maximum(s2[:, C:C + 1], 1.0)
    mm = jax.lax.dot_general(means1[1:], means2[1:], (((1,), (1,)), ((), ())),
                             preferred_element_type=jnp.float32)
    n1 = n_ref[0, 0, 0]
    n2 = n_ref[0, 0, 1]
    k1 = jnp.minimum(LIMIT, n1)
    k2 = jnp.minimum(LIMIT, n2)
    ri = jax.lax.broadcasted_iota(jnp.int32, (LIMIT - 1, LIMIT - 1), 0)
    rj = jax.lax.broadcasted_iota(jnp.int32, (LIMIT - 1, LIMIT - 1), 1)
    valid = (ri < k1 - 1) & (rj < k2 - 1)
    sub = 1.0 + jnp.where(valid, mm, 0.0)
    m1 = jnp.ones((LIMIT - 1, 1), jnp.float32)
    m2 = jnp.ones((1, LIMIT - 1), jnp.float32)
    for j in range(LIMIT - 1):
        m1 = m1 * sub[:, j:j + 1]
        m2 = m2 * sub[j:j + 1, :]
    rr = jax.lax.broadcasted_iota(jnp.int32, (1, LIMIT), 1)
    one = jnp.ones((1, 1), jnp.float32)
    m1p = jnp.concatenate([one, m1.reshape(1, LIMIT - 1)], axis=1)
    m2p = jnp.concatenate([one, m2], axis=1)
    apply = (n1 > 1) & (n2 > 1) & (bg_ref[0, 0, 0] > 0) & (bg_ref[0, 0, 1] > 0)
    mvec1 = jnp.where(apply & (rr >= 1) & (rr < k1), m1p, 1.0)
    mvec2 = jnp.where(apply & (rr >= 1) & (rr < k2), m2p, 1.0)
    mv_ref[0, 0] = mvec1[0]
    mv_ref[0, 1] = mvec2[0]


def _mult_kernel(x_ref, inv_ref, mv_ref, mult_ref, stats_ref):
    @pl.when(pl.program_id(1) == 0)
    def _():
        stats_ref[...] = jnp.zeros_like(stats_ref)

    x = x_ref[0].reshape(C, TH * W)
    rseg = jax.lax.broadcasted_iota(jnp.int32, (LIMIT, TH * W), 0)
    mult = None
    for s in range(2):
        inv = inv_ref[0, s].reshape(1, TH * W)
        oh = (inv == rseg).astype(jnp.float32)
        g = jax.lax.dot_general(mv_ref[0, s].reshape(1, LIMIT), oh,
                                (((1,), (0,)), ((), ())),
                                preferred_element_type=jnp.float32)
        ms = jnp.where(inv < LIMIT, g, 1.0)
        mult = ms if mult is None else mult * ms
    mult_ref[0] = mult.reshape(TH, W)
    xu = x * mult
    stats_ref[0, 0] += jnp.sum(xu, axis=1)
    stats_ref[0, 1] += jnp.sum(xu * xu, axis=1)


def _norm_kernel(x_ref, mult_ref, scale_ref, shift_ref, out_ref):
    xm = x_ref[0] * mult_ref[0][None, :, :]
    out_ref[0] = xm * scale_ref[:][:, :, None] + shift_ref[:][:, :, None]


def kernel(x, conv_w, conv_b, gamma, beta):
    xp = jnp.pad(x, ((0, 0), (0, 0), (1, 31), (1, 1)))

    attn, masks = pl.pallas_call(
        _conv_kernel,
        grid=(B, HT),
        in_specs=[
            pl.BlockSpec((1, C, TH, W + 2), lambda b, h: (b, 0, h, 0)),
            pl.BlockSpec((1, C, TH, W + 2), lambda b, h: (b, 0, h + 1, 0)),
            pl.BlockSpec((3, C, 3, 3), lambda b, h: (0, 0, 0, 0)),
            pl.BlockSpec((3, 1), lambda b, h: (0, 0)),
        ],
        out_specs=[
            pl.BlockSpec((1, 3, TH, W), lambda b, h: (b, 0, h, 0)),
            pl.BlockSpec((1, 2, TH, W), lambda b, h: (b, 0, h, 0)),
        ],
        out_shape=[
            jax.ShapeDtypeStruct((B, 3, H, W), jnp.float32),
            jax.ShapeDtypeStruct((B, 2, H, W), jnp.int32),
        ],
    )(xp, xp, conv_w, conv_b.reshape(3, 1))

    inv, meta = pl.pallas_call(
        _cc_kernel,
        out_shape=[
            jax.ShapeDtypeStruct((2 * B, H, W), jnp.int32),
            jax.ShapeDtypeStruct((8, 128), jnp.int32),
        ],
    )(masks.reshape(2 * B, H, W))
    inv = inv.reshape(B, 2, H, W)
    n_arr = meta[0:4, 0].reshape(B, 1, 2)
    bg_arr = meta[4:8, 0].reshape(B, 1, 2)

    _, mv = pl.pallas_call(
        _segsum_kernel,
        grid=(B, HT),
        in_specs=[
            pl.BlockSpec((1, C, TH, W), lambda b, h: (b, 0, h, 0)),
            pl.BlockSpec((1, 2, TH, W), lambda b, h: (b, 0, h, 0)),
            pl.BlockSpec((1, 1, 2), lambda b, h: (b, 0, 0)),
            pl.BlockSpec((1, 1, 2), lambda b, h: (b, 0, 0)),
        ],
        out_specs=[
            pl.BlockSpec((1, 2, LIMIT, C + 1), lambda b, h: (b, 0, 0, 0)),
            pl.BlockSpec((1, 2, LIMIT), lambda b, h: (b, 0, 0)),
        ],
        out_shape=[
            jax.ShapeDtypeStruct((B, 2, LIMIT, C + 1), jnp.float32),
            jax.ShapeDtypeStruct((B, 2, LIMIT), jnp.float32),
        ],
    )(x, inv, n_arr, bg_arr)

    mult, stats = pl.pallas_call(
        _mult_kernel,
        grid=(B, HT),
        in_specs=[
            pl.BlockSpec((1, C, TH, W), lambda b, h: (b, 0, h, 0)),
            pl.BlockSpec((1, 2, TH, W), lambda b, h: (b, 0, h, 0)),
            pl.BlockSpec((1, 2, LIMIT), lambda b, h: (b, 0, 0)),
        ],
        out_specs=[
            pl.BlockSpec((1, TH, W), lambda b, h: (b, h, 0)),
            pl.BlockSpec((1, 2, C), lambda b, h: (b, 0, 0)),
        ],
        out_shape=[
            jax.ShapeDtypeStruct((B, H, W), jnp.float32),
            jax.ShapeDtypeStruct((B, 2, C), jnp.float32),
        ],
    )(x, inv, mv)

    total = jnp.sum(stats, axis=0)
    n_el = jnp.float32(B * HW)
    mean = total[0] / n_el
    var = total[1] / n_el - mean * mean
    scale = gamma / jnp.sqrt(var + EPS)
    shift = beta - mean * scale

    xn = pl.pallas_call(
        _norm_kernel,
        grid=(B, HT),
        in_specs=[
            pl.BlockSpec((1, C, TH, W), lambda b, h: (b, 0, h, 0)),
            pl.BlockSpec((1, TH, W), lambda b, h: (b, h, 0)),
            pl.BlockSpec((C, 1), lambda b, h: (0, 0)),
            pl.BlockSpec((C, 1), lambda b, h: (0, 0)),
        ],
        out_specs=pl.BlockSpec((1, C, TH, W), lambda b, h: (b, 0, h, 0)),
        out_shape=jax.ShapeDtypeStruct((B, C, H, W), jnp.float32),
    )(x, mult, scale.reshape(C, 1), shift.reshape(C, 1))

    return (xn, attn)


# speedup vs baseline: 1.7499x; 1.4293x over previous
import jax
import jax.numpy as jnp
from jax.experimental import pallas as pl

B, C, H, W = 2, 96, 224, 224
HW = H * W
LIMIT = 20
EPS = 1e-5
TH = 32
HT = H // TH


def _shift(a, axis, d):
    if d == 0:
        return a
    n = a.shape[axis]
    zshape = list(a.shape)
    zshape[axis] = abs(d)
    zeros = jnp.zeros(zshape, a.dtype)
    if d > 0:
        sl = jax.lax.slice_in_dim(a, 0, n - d, axis=axis)
        return jnp.concatenate([zeros, sl], axis=axis)
    sl = jax.lax.slice_in_dim(a, -d, n, axis=axis)
    return jnp.concatenate([sl, zeros], axis=axis)


def _conv_kernel(xa_ref, xb_ref, w_ref, b_ref, attn_ref, mask_ref):
    xx = jnp.concatenate([xa_ref[0], xb_ref[0][:, 0:2]], axis=1)
    pm = jax.lax.dot_general(w_ref[:], xx, (((1,), (0,)), ((), ())),
                             preferred_element_type=jnp.float32)
    rows = []
    for o in range(3):
        r = jnp.zeros((TH, W), jnp.float32)
        for dy in range(3):
            for kx in range(3):
                r = r + pm[o * 9 + dy * 3 + kx, dy:dy + TH, kx:kx + W]
        rows.append(r[None])
    acc = jnp.concatenate(rows, axis=0)
    acc = acc + b_ref[:][:, :, None]
    mx = jnp.max(acc, axis=0, keepdims=True)
    e = jnp.exp(acc - mx)
    attn_ref[0] = e / jnp.sum(e, axis=0, keepdims=True)
    o0, o1, o2 = acc[0], acc[1], acc[2]
    am1 = (o1 > o0) & (o1 >= o2)
    am2 = (o2 > o0) & (o2 > o1)
    mask_ref[0, 0] = am1.astype(jnp.int32)
    mask_ref[0, 1] = am2.astype(jnp.int32)


def _cc_kernel(mask_ref, inv_ref, meta_ref):
    mi = mask_ref[:]
    m = mi != 0
    ih = jax.lax.broadcasted_iota(jnp.int32, (4, H, W), 1)
    iw = jax.lax.broadcasted_iota(jnp.int32, (4, H, W), 2)
    iota = ih * W + iw + 1
    lab0 = jnp.where(m, iota, 0)

    def sweep(lab):
        okf = mi * _shift(mi, 2, 1)
        okb = mi * _shift(mi, 2, -1)
        d = 1
        while d < W:
            lab = jnp.maximum(lab, jnp.where(okf != 0, _shift(lab, 2, d), 0))
            lab = jnp.maximum(lab, jnp.where(okb != 0, _shift(lab, 2, -d), 0))
            okf = okf * _shift(okf, 2, d)
            okb = okb * _shift(okb, 2, -d)
            d *= 2
        okf = mi * _shift(mi, 1, 1)
        okb = mi * _shift(mi, 1, -1)
        d = 1
        while d < H:
            lab = jnp.maximum(lab, jnp.where(okf != 0, _shift(lab, 1, d), 0))
            lab = jnp.maximum(lab, jnp.where(okb != 0, _shift(lab, 1, -d), 0))
            okf = okf * _shift(okf, 1, d)
            okb = okb * _shift(okb, 1, -d)
            d *= 2
        return lab

    def fixpoint(lab):
        state = (lab, sweep(lab))
        state = jax.lax.while_loop(
            lambda s: jnp.any(s[0] != s[1]),
            lambda s: (s[1], sweep(s[1])),
            state)
        return state[1]

    lab = fixpoint(lab0)

    is_head = m & (lab == iota)
    heads = is_head.astype(jnp.int32)
    cs = heads
    d = 1
    while d < W:
        cs = cs + _shift(cs, 2, d)
        d *= 2
    row_tot = cs[:, :, W - 1:W]
    rc = row_tot
    d = 1
    while d < H:
        rc = rc + _shift(rc, 1, d)
        d *= 2
    excl_rows = rc - row_tot
    incl = cs + excl_rows
    excl = incl - heads
    has_bg = jnp.any(jnp.any(~m, axis=2), axis=1)
    bg_i = has_bg.astype(jnp.int32)

    n_heads = incl[:, H - 1, W - 1]
    pos = iota - 1
    inv_cnt = jnp.zeros((4, H, W), jnp.int32)
    for r in range(LIMIT + 1):
        sel = is_head & (excl == r)
        t_sum = jnp.sum(jnp.sum(jnp.where(sel, pos, 0), axis=2), axis=1)
        t_r = jnp.where(r < n_heads, t_sum, HW + 5)
        inv_cnt = inv_cnt + (t_r[:, None, None] < lab - 1).astype(jnp.int32)
    inv_ref[:] = jnp.where(m, bg_i[:, None, None] + inv_cnt, 0)

    n_vec = n_heads + bg_i
    col = jnp.concatenate([n_vec, bg_i], axis=0).reshape(8, 1)
    meta_ref[:] = jnp.concatenate(
        [col, jnp.zeros((8, 127), jnp.int32)], axis=1)


def _segsum_kernel(x_ref, inv_ref, n_ref, bg_ref, sums_ref, mv_ref):
    @pl.when(pl.program_id(1) == 0)
    def _():
        sums_ref[...] = jnp.zeros_like(sums_ref)

    x = x_ref[0].reshape(C, TH * W)
    rseg = jax.lax.broadcasted_iota(jnp.int32, (LIMIT, TH * W), 0)
    for s in range(2):
        inv = inv_ref[0, s].reshape(1, TH * W)
        oh = (inv == rseg).astype(jnp.float32)
        part = jax.lax.dot_general(oh, x, (((1,), (1,)), ((), ())),
                                   preferred_element_type=jnp.float32)
        cnt = jnp.sum(oh, axis=1, keepdims=True)
        sums_ref[0, s] += jnp.concatenate([part, cnt], axis=1)

    @pl.when(pl.program_id(1) == HT - 1)
    def _():
        _factors_body(sums_ref, n_ref, bg_ref, mv_ref)


def _factors_body(sums_ref, n_ref, bg_ref, mv_ref):
    s1 = sums_ref[0, 0]
    s2 = sums_ref[0, 1]
    means1 = s1[:, :C] / jnp.maximum(s1[:, C:C + 1], 1.0)
    means2 = s2[:, :C] / jnp.maximum(s2[:, C:C + 1], 1.0)
    mm = jax.lax.dot_general(means1[1:], means2[1:], (((1,), (1,)), ((), ())),
                             preferred_element_type=jnp.float32)
    n1 = n_ref[0, 0, 0]
    n2 = n_ref[0, 0, 1]
    k1 = jnp.minimum(LIMIT, n1)
    k2 = jnp.minimum(LIMIT, n2)
    ri = jax.lax.broadcasted_iota(jnp.int32, (LIMIT - 1, LIMIT - 1), 0)
    rj = jax.lax.broadcasted_iota(jnp.int32, (LIMIT - 1, LIMIT - 1), 1)
    valid = (ri < k1 - 1) & (rj < k2 - 1)
    sub = 1.0 + jnp.where(valid, mm, 0.0)
    m1 = jnp.ones((LIMIT - 1, 1), jnp.float32)
    m2 = jnp.ones((1, LIMIT - 1), jnp.float32)
    for j in range(LIMIT - 1):
        m1 = m1 * sub[:, j:j + 1]
        m2 = m2 * sub[j:j + 1, :]
    rr = jax.lax.broadcasted_iota(jnp.int32, (1, LIMIT), 1)
    one = jnp.ones((1, 1), jnp.float32)
    m1p = jnp.concatenate([one, m1.reshape(1, LIMIT - 1)], axis=1)
    m2p = jnp.concatenate([one, m2], axis=1)
    apply = (n1 > 1) & (n2 > 1) & (bg_ref[0, 0, 0] > 0) & (bg_ref[0, 0, 1] > 0)
    mvec1 = jnp.where(apply & (rr >= 1) & (rr < k1), m1p, 1.0)
    mvec2 = jnp.where(apply & (rr >= 1) & (rr < k2), m2p, 1.0)
    mv_ref[0, 0] = mvec1[0]
    mv_ref[0, 1] = mvec2[0]


def _mult_kernel(x_ref, inv_ref, mv_ref, mult_ref, stats_ref):
    @pl.when(pl.program_id(1) == 0)
    def _():
        stats_ref[...] = jnp.zeros_like(stats_ref)

    x = x_ref[0].reshape(C, TH * W)
    rseg = jax.lax.broadcasted_iota(jnp.int32, (LIMIT, TH * W), 0)
    mult = None
    for s in range(2):
        inv = inv_ref[0, s].reshape(1, TH * W)
        oh = (inv == rseg).astype(jnp.float32)
        g = jax.lax.dot_general(mv_ref[0, s].reshape(1, LIMIT), oh,
                                (((1,), (0,)), ((), ())),
                                preferred_element_type=jnp.float32)
        ms = jnp.where(inv < LIMIT, g, 1.0)
        mult = ms if mult is None else mult * ms
    mult_ref[0] = mult.reshape(TH, W)
    xu = x * mult
    stats_ref[0, 0] += jnp.sum(xu, axis=1)
    stats_ref[0, 1] += jnp.sum(xu * xu, axis=1)


def _norm_kernel(x_ref, mult_ref, scale_ref, shift_ref, out_ref):
    xm = x_ref[0] * mult_ref[0][None, :, :]
    out_ref[0] = xm * scale_ref[:][:, :, None] + shift_ref[:][:, :, None]


def kernel(x, conv_w, conv_b, gamma, beta):
    xp = jnp.pad(x, ((0, 0), (0, 0), (1, 31), (1, 1)))

    attn, masks = pl.pallas_call(
        _conv_kernel,
        grid=(B, HT),
        in_specs=[
            pl.BlockSpec((1, C, TH, W + 2), lambda b, h: (b, 0, h, 0)),
            pl.BlockSpec((1, C, TH, W + 2), lambda b, h: (b, 0, h + 1, 0)),
            pl.BlockSpec((27, C), lambda b, h: (0, 0)),
            pl.BlockSpec((3, 1), lambda b, h: (0, 0)),
        ],
        out_specs=[
            pl.BlockSpec((1, 3, TH, W), lambda b, h: (b, 0, h, 0)),
            pl.BlockSpec((1, 2, TH, W), lambda b, h: (b, 0, h, 0)),
        ],
        out_shape=[
            jax.ShapeDtypeStruct((B, 3, H, W), jnp.float32),
            jax.ShapeDtypeStruct((B, 2, H, W), jnp.int32),
        ],
    )(xp, xp, conv_w.transpose(0, 2, 3, 1).reshape(27, C), conv_b.reshape(3, 1))

    inv, meta = pl.pallas_call(
        _cc_kernel,
        out_shape=[
            jax.ShapeDtypeStruct((2 * B, H, W), jnp.int32),
            jax.ShapeDtypeStruct((8, 128), jnp.int32),
        ],
    )(masks.reshape(2 * B, H, W))
    inv = inv.reshape(B, 2, H, W)
    n_arr = meta[0:4, 0].reshape(B, 1, 2)
    bg_arr = meta[4:8, 0].reshape(B, 1, 2)

    _, mv = pl.pallas_call(
        _segsum_kernel,
        grid=(B, HT),
        in_specs=[
            pl.BlockSpec((1, C, TH, W), lambda b, h: (b, 0, h, 0)),
            pl.BlockSpec((1, 2, TH, W), lambda b, h: (b, 0, h, 0)),
            pl.BlockSpec((1, 1, 2), lambda b, h: (b, 0, 0)),
            pl.BlockSpec((1, 1, 2), lambda b, h: (b, 0, 0)),
        ],
        out_specs=[
            pl.BlockSpec((1, 2, LIMIT, C + 1), lambda b, h: (b, 0, 0, 0)),
            pl.BlockSpec((1, 2, LIMIT), lambda b, h: (b, 0, 0)),
        ],
        out_shape=[
            jax.ShapeDtypeStruct((B, 2, LIMIT, C + 1), jnp.float32),
            jax.ShapeDtypeStruct((B, 2, LIMIT), jnp.float32),
        ],
    )(x, inv, n_arr, bg_arr)

    mult, stats = pl.pallas_call(
        _mult_kernel,
        grid=(B, HT),
        in_specs=[
            pl.BlockSpec((1, C, TH, W), lambda b, h: (b, 0, h, 0)),
            pl.BlockSpec((1, 2, TH, W), lambda b, h: (b, 0, h, 0)),
            pl.BlockSpec((1, 2, LIMIT), lambda b, h: (b, 0, 0)),
        ],
        out_specs=[
            pl.BlockSpec((1, TH, W), lambda b, h: (b, h, 0)),
            pl.BlockSpec((1, 2, C), lambda b, h: (b, 0, 0)),
        ],
        out_shape=[
            jax.ShapeDtypeStruct((B, H, W), jnp.float32),
            jax.ShapeDtypeStruct((B, 2, C), jnp.float32),
        ],
    )(x, inv, mv)

    total = jnp.sum(stats, axis=0)
    n_el = jnp.float32(B * HW)
    mean = total[0] / n_el
    var = total[1] / n_el - mean * mean
    scale = gamma / jnp.sqrt(var + EPS)
    shift = beta - mean * scale

    xn = pl.pallas_call(
        _norm_kernel,
        grid=(B, HT),
        in_specs=[
            pl.BlockSpec((1, C, TH, W), lambda b, h: (b, 0, h, 0)),
            pl.BlockSpec((1, TH, W), lambda b, h: (b, h, 0)),
            pl.BlockSpec((C, 1), lambda b, h: (0, 0)),
            pl.BlockSpec((C, 1), lambda b, h: (0, 0)),
        ],
        out_specs=pl.BlockSpec((1, C, TH, W), lambda b, h: (b, 0, h, 0)),
        out_shape=jax.ShapeDtypeStruct((B, C, H, W), jnp.float32),
    )(x, mult, scale.reshape(C, 1), shift.reshape(C, 1))

    return (xn, attn)
